# baseline (speedup 1.0000x reference)
import functools

import jax
import jax.numpy as jnp
from jax import lax
from jax.experimental import pallas as pl
from jax.experimental.pallas import tpu as pltpu
from jax.experimental.pallas import tpu_sc as plsc

DEP = 128
SLOT = 128
GROUPS = 2
GROUP_SIZE = 1024
TOTAL = GROUPS * GROUP_SIZE

NC = 2
NS = 16
NW = NC * NS
D_PER_W = DEP // NW

CH = 128
NBUF = 3
M_SC = 3
SC_ITEMS = M_SC * CH
BPG = SC_ITEMS // 16
NDG = D_PER_W * GROUPS
NCHTOT = NDG * M_SC
NB16 = NDG * BPG
CHB = CH // 16

DB_TC = 32
TC_ITEMS = GROUP_SIZE - SC_ITEMS


def _bcast_idx(i):
    return jnp.full((16,), i, jnp.int32)


_GDN = lax.GatherDimensionNumbers(
    offset_dims=(), collapsed_slice_dims=(0,), start_index_map=(0,))


def _lane_bcast(v16, l):
    return lax.gather(v16, _bcast_idx(l)[:, None], _GDN, slice_sizes=(1,),
                      mode=lax.GatherScatterMode.PROMISE_IN_BOUNDS)


def _sc_body(addr_hbm, embt_hbm, freq_hbm, out_hbm,
             freq_v, fd_all, et_all, abuf, ov, sem0, sem1, sem2):
    wid = lax.axis_index("s") * NC + lax.axis_index("c")
    dd0 = wid * D_PER_W
    sems = (sem0, sem1, sem2)

    def chunk_src(gc):
        idx_dg = lax.div(gc, M_SC)
        c = lax.rem(gc, M_SC)
        dl = lax.div(idx_dg, GROUPS)
        g = lax.rem(idx_dg, GROUPS)
        return addr_hbm.at[dd0 + dl, pl.ds(g * GROUP_SIZE + c * CH, CH)]

    def issue(gc, b):
        pltpu.make_async_copy(
            chunk_src(gc), abuf.at[pl.ds(b * CH, CH)], sems[b]).start()

    pltpu.sync_copy(freq_hbm, freq_v)
    pltpu.sync_copy(embt_hbm.at[pl.ds(dd0, D_PER_W)], et_all)
    issue(0, 0)
    issue(1, 1)

    def fd_body(t, _):
        dl = lax.div(t, TOTAL // 16)
        o = lax.rem(t, TOTAL // 16) * 16
        fd_all[dl, pl.ds(o, 16)] = et_all[dl, pl.ds(o, 16)] * freq_v[pl.ds(o, 16)]
        return 0

    lax.fori_loop(0, D_PER_W * (TOTAL // 16), fd_body, 0)

    def blk_body(i16, acc):
        gc = lax.div(i16, CHB)
        at_chunk = lax.rem(i16, CHB) == 0
        for b in range(NBUF):
            @pl.when(jnp.logical_and(at_chunk, lax.rem(gc, NBUF) == b))
            def _ring(b=b):
                b2 = (b + 2) % NBUF

                @pl.when(gc + 2 < NCHTOT)
                def _issue():
                    issue(gc + 2, b2)

                pltpu.make_async_copy(
                    addr_hbm.at[0, pl.ds(0, CH)],
                    abuf.at[pl.ds(b * CH, CH)], sems[b]).wait()

        idx_dg = lax.div(i16, BPG)
        loc = lax.rem(i16, BPG)
        at_dg = loc == 0

        @pl.when(jnp.logical_and(at_dg, i16 > 0))
        def _flush():
            pdg = idx_dg - 1
            for j in range(8):
                ov[pl.ds(pdg * SLOT + j * 16, 16)] = acc[j]

        zero = jnp.zeros((16,), jnp.float32)
        acc = [jnp.where(at_dg, zero, a) for a in acc]

        dl = lax.div(idx_dg, GROUPS)
        g = lax.rem(idx_dg, GROUPS)
        f16 = fd_all[dl, pl.ds(g * GROUP_SIZE + loc * 16, 16)]
        r0 = lax.rem(gc, NBUF) * CH + lax.rem(i16, CHB) * 16
        for l in range(16):
            fbc = _lane_bcast(f16, l)
            for j in range(8):
                acc[j] = acc[j] + fbc * abuf[r0 + l, pl.ds(j * 16, 16)]
        return tuple(acc)

    zero = jnp.zeros((16,), jnp.float32)
    acc = lax.fori_loop(0, NB16, blk_body, (zero,) * 8)
    for j in range(8):
        ov[pl.ds((NDG - 1) * SLOT + j * 16, 16)] = acc[j]

    for dl in range(D_PER_W):
        for g in range(GROUPS):
            pltpu.sync_copy(ov.at[pl.ds((dl * GROUPS + g) * SLOT, SLOT)],
                            out_hbm.at[g, dd0 + dl])


def _sc_part(batch_address, embt, batch_frequency):
    mesh = plsc.VectorSubcoreMesh(core_axis_name="c", subcore_axis_name="s")
    f = functools.partial(
        pl.kernel,
        mesh=mesh,
        out_type=jax.ShapeDtypeStruct((GROUPS, DEP, SLOT), jnp.float32),
        scratch_types=[
            pltpu.VMEM((TOTAL,), jnp.float32),
            pltpu.VMEM((D_PER_W, TOTAL), jnp.float32),
            pltpu.VMEM((D_PER_W, TOTAL), jnp.float32),
            pltpu.VMEM((NBUF * CH, SLOT), jnp.float32),
            pltpu.VMEM((NDG * SLOT,), jnp.float32),
            pltpu.SemaphoreType.DMA,
            pltpu.SemaphoreType.DMA,
            pltpu.SemaphoreType.DMA,
        ],
    )(_sc_body)
    return f(batch_address, embt, batch_frequency)


def _tc_body(addr_ref, embt_ref, freq_ref, mem_ref, out_ref):
    k = pl.program_id(1)
    a = addr_ref[...]
    ft = embt_ref[...] * freq_ref[...]
    contrib = jnp.sum(a * ft[:, :, None], axis=1)

    @pl.when(k == 0)
    def _init():
        out_ref[...] = mem_ref[...] + contrib[None]

    @pl.when(k != 0)
    def _acc():
        out_ref[...] += contrib[None]


IB_TC = 128
NIB_TC = TC_ITEMS // IB_TC


def _tc_part(batch_address, embt, freq2d, memory_matrix):
    off = SC_ITEMS // IB_TC
    npg = GROUP_SIZE // IB_TC
    grid = (GROUPS, NIB_TC)
    return pl.pallas_call(
        _tc_body,
        grid=grid,
        in_specs=[
            pl.BlockSpec((DEP, IB_TC, SLOT), lambda g, k: (0, g * npg + off + k, 0)),
            pl.BlockSpec((DEP, IB_TC), lambda g, k: (0, g * npg + off + k)),
            pl.BlockSpec((1, IB_TC), lambda g, k: (0, g * npg + off + k)),
            pl.BlockSpec((1, DEP, SLOT), lambda g, k: (g, 0, 0)),
        ],
        out_specs=pl.BlockSpec((1, DEP, SLOT), lambda g, k: (g, 0, 0)),
        out_shape=jax.ShapeDtypeStruct((GROUPS, DEP, SLOT), jnp.float32),
        compiler_params=pltpu.CompilerParams(
            dimension_semantics=("arbitrary", "arbitrary"),
        ),
    )(batch_address, embt, freq2d, memory_matrix)


def kernel(batch_address, batch_embedding, batch_frequency, memory_matrix):
    embt = batch_embedding.T
    sc_out = _sc_part(batch_address, embt, batch_frequency)
    tc_out = _tc_part(batch_address, embt, batch_frequency[None, :],
                      memory_matrix)
    return sc_out + tc_out


# speedup vs baseline: 1.3007x; 1.3007x over previous
import jax
import jax.numpy as jnp
from jax.experimental import pallas as pl
from jax.experimental.pallas import tpu as pltpu

DEP = 128
SLOT = 128
GROUPS = 2
GROUP_SIZE = 1024
DB = 32


def _body(addr_ref, f_ref, mem_ref, out_ref, fts_ref):
    db = pl.program_id(1)

    @pl.when(db == 0)
    def _tr():
        fts_ref[...] = f_ref[...].T

    a = addr_ref[...]
    ft = fts_ref[pl.ds(db * DB, DB), :]
    contrib = jnp.sum(a * ft[:, :, None], axis=1)
    out_ref[...] = mem_ref[...] + contrib[None]


def kernel(batch_address, batch_embedding, batch_frequency, memory_matrix):
    f_emb = batch_embedding * batch_frequency[:, None]
    n_db = DEP // DB
    grid = (GROUPS, n_db)
    return pl.pallas_call(
        _body,
        grid=grid,
        in_specs=[
            pl.BlockSpec((DB, GROUP_SIZE, SLOT), lambda g, db: (db, g, 0)),
            pl.BlockSpec((GROUP_SIZE, DEP), lambda g, db: (g, 0)),
            pl.BlockSpec((1, DB, SLOT), lambda g, db: (g, db, 0)),
        ],
        out_specs=pl.BlockSpec((1, DB, SLOT), lambda g, db: (g, db, 0)),
        out_shape=jax.ShapeDtypeStruct((GROUPS, DEP, SLOT), jnp.float32),
        scratch_shapes=[pltpu.VMEM((DEP, GROUP_SIZE), jnp.float32)],
        compiler_params=pltpu.CompilerParams(
            dimension_semantics=("arbitrary", "arbitrary"),
        ),
    )(batch_address, f_emb, memory_matrix)


# speedup vs baseline: 1.3852x; 1.0650x over previous
import jax
import jax.numpy as jnp
from jax import lax
from jax.experimental import pallas as pl
from jax.experimental.pallas import tpu as pltpu

DEP = 128
SLOT = 128
GROUPS = 2
GROUP_SIZE = 1024
TOTAL = GROUPS * GROUP_SIZE
CHD = 8
NCH = DEP // CHD
NBUF = 3


def _body(addr_hbm, f_ref, mem_ref, out_ref, abuf, fts, sem0, sem1, sem2):
    sems = (sem0, sem1, sem2)

    def copy(c, b):
        return pltpu.make_async_copy(
            addr_hbm.at[pl.ds(c * CHD, CHD)],
            abuf.at[pl.ds(b * CHD, CHD)],
            sems[b])

    copy(0, 0).start()
    copy(1, 1).start()
    fts[...] = f_ref[...].T

    def chunk(c, _):
        b = lax.rem(c, NBUF)
        for bb in range(NBUF):
            @pl.when(jnp.logical_and(b == bb, c + 2 < NCH))
            def _issue(bb=bb):
                copy(c + 2, (bb + 2) % NBUF).start()

            @pl.when(b == bb)
            def _wait(bb=bb):
                copy(0, bb).wait()

        a = abuf[pl.ds(b * CHD, CHD)]
        ftc = fts[pl.ds(c * CHD, CHD), :]
        c0 = jnp.sum(a[:, :GROUP_SIZE, :] * ftc[:, :GROUP_SIZE, None], axis=1)
        c1 = jnp.sum(a[:, GROUP_SIZE:, :] * ftc[:, GROUP_SIZE:, None], axis=1)
        out_ref[0, pl.ds(c * CHD, CHD), :] = mem_ref[0, pl.ds(c * CHD, CHD), :] + c0
        out_ref[1, pl.ds(c * CHD, CHD), :] = mem_ref[1, pl.ds(c * CHD, CHD), :] + c1
        return 0

    lax.fori_loop(0, NCH, chunk, 0)


def kernel(batch_address, batch_embedding, batch_frequency, memory_matrix):
    f_emb = batch_embedding * batch_frequency[:, None]
    return pl.pallas_call(
        _body,
        in_specs=[
            pl.BlockSpec(memory_space=pltpu.MemorySpace.HBM),
            pl.BlockSpec((TOTAL, DEP), lambda: (0, 0)),
            pl.BlockSpec((GROUPS, DEP, SLOT), lambda: (0, 0, 0)),
        ],
        out_specs=pl.BlockSpec((GROUPS, DEP, SLOT), lambda: (0, 0, 0)),
        out_shape=jax.ShapeDtypeStruct((GROUPS, DEP, SLOT), jnp.float32),
        scratch_shapes=[
            pltpu.VMEM((NBUF * CHD, TOTAL, SLOT), jnp.float32),
            pltpu.VMEM((DEP, TOTAL), jnp.float32),
            pltpu.SemaphoreType.DMA,
            pltpu.SemaphoreType.DMA,
            pltpu.SemaphoreType.DMA,
        ],
        compiler_params=pltpu.CompilerParams(
            vmem_limit_bytes=100 * 1024 * 1024,
        ),
    )(batch_address, f_emb, memory_matrix)


# speedup vs baseline: 1.4866x; 1.0732x over previous
import jax
import jax.numpy as jnp
from jax import lax
from jax.experimental import pallas as pl
from jax.experimental.pallas import tpu as pltpu

DEP = 128
SLOT = 128
GROUPS = 2
GROUP_SIZE = 1024
TOTAL = GROUPS * GROUP_SIZE
CHD = 8
NCH = DEP // CHD
NBUF = 3


def _body(addr_hbm, emb_ref, freq_ref, mem_ref, out_ref, abuf, fts, sem0, sem1, sem2):
    sems = (sem0, sem1, sem2)

    def copy(c, b):
        return pltpu.make_async_copy(
            addr_hbm.at[pl.ds(c * CHD, CHD)],
            abuf.at[pl.ds(b * CHD, CHD)],
            sems[b])

    copy(0, 0).start()
    copy(1, 1).start()
    fts[...] = emb_ref[...].T * freq_ref[...]

    def chunk(c, _):
        b = lax.rem(c, NBUF)
        for bb in range(NBUF):
            @pl.when(jnp.logical_and(b == bb, c + 2 < NCH))
            def _issue(bb=bb):
                copy(c + 2, (bb + 2) % NBUF).start()

            @pl.when(b == bb)
            def _wait(bb=bb):
                copy(0, bb).wait()

        a = abuf[pl.ds(b * CHD, CHD)]
        ftc = fts[pl.ds(c * CHD, CHD), :]
        c0 = jnp.sum(a[:, :GROUP_SIZE, :] * ftc[:, :GROUP_SIZE, None], axis=1)
        c1 = jnp.sum(a[:, GROUP_SIZE:, :] * ftc[:, GROUP_SIZE:, None], axis=1)
        out_ref[0, pl.ds(c * CHD, CHD), :] = mem_ref[0, pl.ds(c * CHD, CHD), :] + c0
        out_ref[1, pl.ds(c * CHD, CHD), :] = mem_ref[1, pl.ds(c * CHD, CHD), :] + c1
        return 0

    lax.fori_loop(0, NCH, chunk, 0)


def kernel(batch_address, batch_embedding, batch_frequency, memory_matrix):
    return pl.pallas_call(
        _body,
        in_specs=[
            pl.BlockSpec(memory_space=pltpu.MemorySpace.HBM),
            pl.BlockSpec((TOTAL, DEP), lambda: (0, 0)),
            pl.BlockSpec((1, TOTAL), lambda: (0, 0)),
            pl.BlockSpec((GROUPS, DEP, SLOT), lambda: (0, 0, 0)),
        ],
        out_specs=pl.BlockSpec((GROUPS, DEP, SLOT), lambda: (0, 0, 0)),
        out_shape=jax.ShapeDtypeStruct((GROUPS, DEP, SLOT), jnp.float32),
        scratch_shapes=[
            pltpu.VMEM((NBUF * CHD, TOTAL, SLOT), jnp.float32),
            pltpu.VMEM((DEP, TOTAL), jnp.float32),
            pltpu.SemaphoreType.DMA,
            pltpu.SemaphoreType.DMA,
            pltpu.SemaphoreType.DMA,
        ],
        compiler_params=pltpu.CompilerParams(
            vmem_limit_bytes=100 * 1024 * 1024,
        ),
    )(batch_address, batch_embedding, batch_frequency[None, :], memory_matrix)
